# Initial kernel scaffold; baseline (speedup 1.0000x reference)
import functools

import jax
import jax.numpy as jnp
from jax import lax
from jax.experimental import pallas as pl
from jax.experimental.pallas import tpu as pltpu
from jax.experimental.pallas import tpu_sc as plsc

N = 320000
D = 128
NUM_SEG = 10000

NC = 2
NS = 16
NW = NC * NS

ROWS_PER_W = N // NW
CHUNK = 400
GRP = 80
GRPS_PER_CHUNK = CHUNK // GRP
CHUNKS = ROWS_PER_W // CHUNK
STRIPE = NUM_SEG // NS


def _sc_segment_partials(x, batch2d, zeros_stripe):
    mesh = plsc.VectorSubcoreMesh(core_axis_name="c", subcore_axis_name="s")

    @functools.partial(
        pl.kernel,
        mesh=mesh,
        out_type=jax.ShapeDtypeStruct((NC, NUM_SEG, D), jnp.float32),
        scratch_types=[
            pltpu.VMEM((CHUNK, D), jnp.float32),
            pltpu.VMEM((GRPS_PER_CHUNK, GRP), jnp.int32),
            pltpu.VMEM_SHARED((NUM_SEG, D), jnp.float32),
        ],
    )
    def k(x_hbm, b_hbm, z_hbm, out_hbm, rows_v, idx_v, acc):
        cid = lax.axis_index("c")
        sid = lax.axis_index("s")
        wid = sid * NC + cid

        pltpu.sync_copy(z_hbm, acc.at[pl.ds(sid * STRIPE, STRIPE)])
        plsc.subcore_barrier()

        base_row = wid * ROWS_PER_W
        base_grp = base_row // GRP

        def body(c, carry):
            row0 = base_row + c * CHUNK
            g0 = base_grp + c * GRPS_PER_CHUNK
            pltpu.sync_copy(b_hbm.at[pl.ds(g0, GRPS_PER_CHUNK)], idx_v)
            pltpu.sync_copy(x_hbm.at[pl.ds(row0, CHUNK)], rows_v)
            for j in range(GRPS_PER_CHUNK):
                pltpu.sync_copy(
                    rows_v.at[pl.ds(j * GRP, GRP)],
                    acc.at[idx_v.at[j]],
                    add=True,
                )
            return carry

        lax.fori_loop(0, CHUNKS, body, 0)
        plsc.subcore_barrier()

        pltpu.sync_copy(
            acc.at[pl.ds(sid * STRIPE, STRIPE)],
            out_hbm.at[cid].at[pl.ds(sid * STRIPE, STRIPE)],
        )

    return k(x, batch2d, zeros_stripe)


def _add_partials(partials):
    def body(a_ref, b_ref, o_ref):
        o_ref[...] = a_ref[0] + b_ref[0]

    blk = 1000
    return pl.pallas_call(
        body,
        grid=(NUM_SEG // blk,),
        in_specs=[
            pl.BlockSpec((1, blk, D), lambda i: (0, i, 0)),
            pl.BlockSpec((1, blk, D), lambda i: (1, i, 0)),
        ],
        out_specs=pl.BlockSpec((blk, D), lambda i: (i, 0)),
        out_shape=jax.ShapeDtypeStruct((NUM_SEG, D), jnp.float32),
    )(partials, partials)


@jax.jit
def kernel(x, batch):
    batch2d = batch.astype(jnp.int32).reshape(N // GRP, GRP)
    zeros_stripe = jnp.zeros((STRIPE, D), jnp.float32)
    partials = _sc_segment_partials(x, batch2d, zeros_stripe)
    return _add_partials(partials)


# speedup vs baseline: 4.7845x; 4.7845x over previous
import functools

import jax
import jax.numpy as jnp
from jax import lax
from jax.experimental import pallas as pl
from jax.experimental.pallas import tpu as pltpu
from jax.experimental.pallas import tpu_sc as plsc

N = 320000
D = 128
NUM_SEG = 10000

NC = 2
NS = 16
NW = NC * NS

CHUNK = 256
GRP = 32
GRPS_PER_CHUNK = CHUNK // GRP
NCHUNKS = N // CHUNK
STRIPE = 624
STRIPE_LAST = NUM_SEG - (NS - 1) * STRIPE


def _sc_segment_partials(x, batch2d, zeros_stripe):
    mesh = plsc.VectorSubcoreMesh(core_axis_name="c", subcore_axis_name="s")

    @functools.partial(
        pl.kernel,
        mesh=mesh,
        out_type=jax.ShapeDtypeStruct((NC, NUM_SEG, D), jnp.float32),
        scratch_types=[
            pltpu.VMEM((CHUNK, D), jnp.float32),
            pltpu.VMEM((GRPS_PER_CHUNK, GRP), jnp.int32),
            pltpu.VMEM_SHARED((NUM_SEG, D), jnp.float32),
        ],
    )
    def k(x_hbm, b_hbm, z_hbm, out_hbm, rows_v, idx_v, acc):
        cid = lax.axis_index("c")
        sid = lax.axis_index("s")
        wid = sid * NC + cid

        @pl.when(sid < NS - 1)
        def _():
            pltpu.sync_copy(z_hbm.at[pl.ds(0, STRIPE)],
                            acc.at[pl.ds(sid * STRIPE, STRIPE)])

        @pl.when(sid == NS - 1)
        def _():
            pltpu.sync_copy(z_hbm,
                            acc.at[pl.ds((NS - 1) * STRIPE, STRIPE_LAST)])

        plsc.subcore_barrier()

        nchunks_w = (NCHUNKS - wid + NW - 1) // NW

        def body(i, carry):
            c = wid + i * NW
            row0 = c * CHUNK
            g0 = c * GRPS_PER_CHUNK
            pltpu.sync_copy(b_hbm.at[pl.ds(g0, GRPS_PER_CHUNK)], idx_v)
            pltpu.sync_copy(x_hbm.at[pl.ds(row0, CHUNK)], rows_v)
            for j in range(GRPS_PER_CHUNK):
                pltpu.sync_copy(
                    rows_v.at[pl.ds(j * GRP, GRP)],
                    acc.at[idx_v.at[j]],
                    add=True,
                )
            return carry

        lax.fori_loop(0, nchunks_w, body, 0)
        plsc.subcore_barrier()

        @pl.when(sid < NS - 1)
        def _():
            pltpu.sync_copy(
                acc.at[pl.ds(sid * STRIPE, STRIPE)],
                out_hbm.at[cid].at[pl.ds(sid * STRIPE, STRIPE)],
            )

        @pl.when(sid == NS - 1)
        def _():
            pltpu.sync_copy(
                acc.at[pl.ds((NS - 1) * STRIPE, STRIPE_LAST)],
                out_hbm.at[cid].at[pl.ds((NS - 1) * STRIPE, STRIPE_LAST)],
            )

    return k(x, batch2d, zeros_stripe)


def _add_partials(partials):
    def body(a_ref, b_ref, o_ref):
        o_ref[...] = a_ref[0] + b_ref[0]

    blk = 1000
    return pl.pallas_call(
        body,
        grid=(NUM_SEG // blk,),
        in_specs=[
            pl.BlockSpec((1, blk, D), lambda i: (0, i, 0)),
            pl.BlockSpec((1, blk, D), lambda i: (1, i, 0)),
        ],
        out_specs=pl.BlockSpec((blk, D), lambda i: (i, 0)),
        out_shape=jax.ShapeDtypeStruct((NUM_SEG, D), jnp.float32),
    )(partials, partials)


@jax.jit
def kernel(x, batch):
    batch2d = batch.astype(jnp.int32).reshape(N // GRP, GRP)
    zeros_stripe = jnp.zeros((STRIPE_LAST, D), jnp.float32)
    partials = _sc_segment_partials(x, batch2d, zeros_stripe)
    return _add_partials(partials)


# speedup vs baseline: 7.1492x; 1.4942x over previous
import functools

import jax
import jax.numpy as jnp
from jax import lax
from jax.experimental import pallas as pl
from jax.experimental.pallas import tpu as pltpu
from jax.experimental.pallas import tpu_sc as plsc

N = 320000
D = 128
NUM_SEG = 10000

NC = 2
NS = 16
NW = NC * NS

CHUNK = 128
GRP = 16
GRPS_PER_CHUNK = CHUNK // GRP
NCHUNKS = N // CHUNK
STRIPE = 624
STRIPE_LAST = NUM_SEG - (NS - 1) * STRIPE


def _sc_segment_partials(x, batch2d, zeros_stripe):
    mesh = plsc.VectorSubcoreMesh(core_axis_name="c", subcore_axis_name="s")

    @functools.partial(
        pl.kernel,
        mesh=mesh,
        out_type=jax.ShapeDtypeStruct((NC, NUM_SEG, D), jnp.float32),
        scratch_types=[
            pltpu.VMEM((CHUNK, D), jnp.float32),
            pltpu.VMEM((CHUNK, D), jnp.float32),
            pltpu.VMEM((GRPS_PER_CHUNK, GRP), jnp.int32),
            pltpu.VMEM((GRPS_PER_CHUNK, GRP), jnp.int32),
            pltpu.VMEM_SHARED((NUM_SEG, D), jnp.float32),
            pltpu.SemaphoreType.DMA,
            pltpu.SemaphoreType.DMA,
            pltpu.SemaphoreType.DMA,
        ],
    )
    def k(x_hbm, b_hbm, z_hbm, out_hbm, rows0, rows1, idx0, idx1, acc,
          sem0, sem1, sem_sc):
        cid = lax.axis_index("c")
        sid = lax.axis_index("s")
        wid = sid * NC + cid

        @pl.when(sid < NS - 1)
        def _():
            pltpu.sync_copy(z_hbm.at[pl.ds(0, STRIPE)],
                            acc.at[pl.ds(sid * STRIPE, STRIPE)])

        @pl.when(sid == NS - 1)
        def _():
            pltpu.sync_copy(z_hbm,
                            acc.at[pl.ds((NS - 1) * STRIPE, STRIPE_LAST)])

        plsc.subcore_barrier()

        nchunks_w = (NCHUNKS - wid + NW - 1) // NW
        slots = ((rows0, idx0, sem0), (rows1, idx1, sem1))

        def fill(slot, i):
            rows_v, idx_v, sem = slot
            c = wid + i * NW
            pltpu.async_copy(x_hbm.at[pl.ds(c * CHUNK, CHUNK)], rows_v, sem)
            pltpu.async_copy(
                b_hbm.at[pl.ds(c * GRPS_PER_CHUNK, GRPS_PER_CHUNK)], idx_v, sem)

        def wait_fill(slot, i):
            rows_v, idx_v, sem = slot
            c = wid + i * NW
            pltpu.make_async_copy(
                x_hbm.at[pl.ds(c * CHUNK, CHUNK)], rows_v, sem).wait()
            pltpu.make_async_copy(
                b_hbm.at[pl.ds(c * GRPS_PER_CHUNK, GRPS_PER_CHUNK)], idx_v,
                sem).wait()

        def scatter(slot):
            rows_v, idx_v, _ = slot
            hs = [
                pltpu.async_copy(
                    rows_v.at[pl.ds(j * GRP, GRP)],
                    acc.at[idx_v.at[j]],
                    sem_sc,
                    add=True,
                )
                for j in range(GRPS_PER_CHUNK)
            ]
            for h in hs:
                h.wait()

        fill(slots[0], 0)
        fill(slots[1], 1)

        def body(p, carry):
            for b in (0, 1):
                i = 2 * p + b

                @pl.when(i < nchunks_w)
                def _():
                    wait_fill(slots[b], i)
                    scatter(slots[b])

                    @pl.when(i + 2 < nchunks_w)
                    def _():
                        fill(slots[b], i + 2)

            return carry

        lax.fori_loop(0, (nchunks_w + 1) // 2, body, 0)
        plsc.subcore_barrier()

        @pl.when(sid < NS - 1)
        def _():
            pltpu.sync_copy(
                acc.at[pl.ds(sid * STRIPE, STRIPE)],
                out_hbm.at[cid].at[pl.ds(sid * STRIPE, STRIPE)],
            )

        @pl.when(sid == NS - 1)
        def _():
            pltpu.sync_copy(
                acc.at[pl.ds((NS - 1) * STRIPE, STRIPE_LAST)],
                out_hbm.at[cid].at[pl.ds((NS - 1) * STRIPE, STRIPE_LAST)],
            )

    return k(x, batch2d, zeros_stripe)


def _add_partials(partials):
    def body(a_ref, b_ref, o_ref):
        o_ref[...] = a_ref[0] + b_ref[0]

    blk = 1000
    return pl.pallas_call(
        body,
        grid=(NUM_SEG // blk,),
        in_specs=[
            pl.BlockSpec((1, blk, D), lambda i: (0, i, 0)),
            pl.BlockSpec((1, blk, D), lambda i: (1, i, 0)),
        ],
        out_specs=pl.BlockSpec((blk, D), lambda i: (i, 0)),
        out_shape=jax.ShapeDtypeStruct((NUM_SEG, D), jnp.float32),
    )(partials, partials)


@jax.jit
def kernel(x, batch):
    batch2d = batch.astype(jnp.int32).reshape(N // GRP, GRP)
    zeros_stripe = jnp.zeros((STRIPE_LAST, D), jnp.float32)
    partials = _sc_segment_partials(x, batch2d, zeros_stripe)
    return _add_partials(partials)
